# Initial kernel scaffold; baseline (speedup 1.0000x reference)
import jax
import jax.numpy as jnp
from jax.experimental import pallas as pl


def kernel(f0, f1, f2, f3, f4, f5, f6, f7, f8, f9, f10, f11, f12, f13, f14, f15, f16, f17, f18, f19, f20, f21, f22, f23, f24, f25, W_f0, W_f1, W_f2, W_f3, W_f4, W_f5, W_f6, W_f7, W_f8, W_f9, W_f10, W_f11, W_f12, W_f13, W_f14, W_f15, W_f16, W_f17, W_f18, W_f19, W_f20, W_f21, W_f22, W_f23, W_f24, W_f25):
    raise NotImplementedError("write your pallas kernel here")


# speedup vs baseline: 2.2799x; 2.2799x over previous
import functools

import jax
import jax.numpy as jnp
from jax import lax
from jax.experimental import pallas as pl
from jax.experimental.pallas import tpu as pltpu
from jax.experimental.pallas import tpu_sc as plsc

NFIELDS = 26
VOCAB = 100000
EMB = 32
B = 4096
L = 20

NC = 2
NS = 16
NW = NC * NS
BPW = B // NW
CL = 128
HC = 10
HALF = HC * CL

GROUPS = (7, 7, 6, 6)

_mesh = plsc.VectorSubcoreMesh(
    core_axis_name="c", subcore_axis_name="s", num_cores=NC, num_subcores=NS
)

_SCRATCH = [
    pltpu.VMEM((HC, CL), jnp.int32),
    pltpu.VMEM((HC, CL), jnp.int32),
    pltpu.VMEM((HC, CL), jnp.int32),
    pltpu.VMEM((HC, CL), jnp.int32),
    pltpu.VMEM((HALF, EMB), jnp.float32),
    pltpu.VMEM((HALF, EMB), jnp.float32),
    pltpu.SemaphoreType.DMA,
    pltpu.SemaphoreType.DMA,
    pltpu.SemaphoreType.DMA,
    pltpu.SemaphoreType.DMA,
]


def _make_body(gsize):
    nunits = 2 * gsize
    nrows = B * L * gsize

    def body(*refs):
        idx_refs = refs[0:gsize]
        w_refs = refs[gsize:2 * gsize]
        dsc_ref = refs[2 * gsize]
        out_ref = refs[2 * gsize + 1]
        sc = refs[2 * gsize + 2:]
        gidx = sc[0:2]
        didx = sc[2:4]
        rows = sc[4:6]
        gsem = sc[6:8]
        ssem = sc[8:10]

        wid = lax.axis_index("s") * NC + lax.axis_index("c")

        def drain_scatter(p):
            pltpu.make_async_copy(
                rows[p], out_ref.at[pl.ds(0, HALF)], ssem[p]
            ).wait()

        for u in range(nunits):
            f, h = divmod(u, 2)
            p = u & 1
            if u >= 2:
                drain_scatter(p)
            pltpu.sync_copy(idx_refs[f].at[wid, pl.ds(h * HC, HC)], gidx[p])
            pltpu.sync_copy(dsc_ref.at[f, wid, pl.ds(h * HC, HC)], didx[p])

            def gbody(j, carry, f=f, p=p):
                pltpu.async_copy(
                    w_refs[f].at[gidx[p].at[j]],
                    rows[p].at[pl.ds(j * CL, CL)],
                    gsem[p],
                )
                return carry

            lax.fori_loop(0, HC, gbody, 0)
            pltpu.make_async_copy(
                out_ref.at[pl.ds(0, HALF)], rows[p], gsem[p]
            ).wait()

            def sbody(j, carry, p=p):
                pltpu.async_copy(
                    rows[p].at[pl.ds(j * CL, CL)],
                    out_ref.at[didx[p].at[j]],
                    ssem[p],
                )
                return carry

            lax.fori_loop(0, HC, sbody, 0)

        drain_scatter(nunits % 2)
        drain_scatter((nunits + 1) % 2)

    return body, jax.ShapeDtypeStruct((nrows, EMB), jnp.float32)


def _make_group_call(gsize):
    body, out_type = _make_body(gsize)
    return pl.kernel(
        body,
        out_type=out_type,
        mesh=_mesh,
        compiler_params=pltpu.CompilerParams(use_tc_tiling_on_sc=False),
        scratch_types=_SCRATCH,
    )


_GROUP_CALLS = {g: _make_group_call(g) for g in set(GROUPS)}


def _dest_rows(gsize):
    q = jnp.arange(B * L, dtype=jnp.int32)
    base = (q // L) * (gsize * L) + (q % L)
    dsc = base[None, :] + (jnp.arange(gsize, dtype=jnp.int32) * L)[:, None]
    return dsc.reshape(gsize, NW, 2 * HC, CL)


def kernel(f0, f1, f2, f3, f4, f5, f6, f7, f8, f9, f10, f11, f12, f13, f14,
           f15, f16, f17, f18, f19, f20, f21, f22, f23, f24, f25,
           W_f0, W_f1, W_f2, W_f3, W_f4, W_f5, W_f6, W_f7, W_f8, W_f9, W_f10,
           W_f11, W_f12, W_f13, W_f14, W_f15, W_f16, W_f17, W_f18, W_f19,
           W_f20, W_f21, W_f22, W_f23, W_f24, W_f25):
    idxs = [f0, f1, f2, f3, f4, f5, f6, f7, f8, f9, f10, f11, f12, f13, f14,
            f15, f16, f17, f18, f19, f20, f21, f22, f23, f24, f25]
    tables = [W_f0, W_f1, W_f2, W_f3, W_f4, W_f5, W_f6, W_f7, W_f8, W_f9,
              W_f10, W_f11, W_f12, W_f13, W_f14, W_f15, W_f16, W_f17, W_f18,
              W_f19, W_f20, W_f21, W_f22, W_f23, W_f24, W_f25]
    pieces = []
    pos = 0
    for gsize in GROUPS:
        gidx = [x.astype(jnp.int32).reshape(NW, 2 * HC, CL)
                for x in idxs[pos:pos + gsize]]
        gtab = tables[pos:pos + gsize]
        out_flat = _GROUP_CALLS[gsize](*gidx, *gtab, _dest_rows(gsize))
        pieces.append(out_flat.reshape(B, gsize * L, EMB))
        pos += gsize
    return jnp.concatenate(pieces, axis=1)


# speedup vs baseline: 2.4784x; 1.0870x over previous
import functools

import jax
import jax.numpy as jnp
from jax import lax
from jax.experimental import pallas as pl
from jax.experimental.pallas import tpu as pltpu
from jax.experimental.pallas import tpu_sc as plsc

NFIELDS = 26
VOCAB = 100000
EMB = 32
B = 4096
L = 20

NC = 2
NS = 16
NW = NC * NS
BPW = B // NW
CL = 128
HC = 10
HALF = HC * CL
NUNITS = 2 * NFIELDS

_mesh = plsc.VectorSubcoreMesh(
    core_axis_name="c", subcore_axis_name="s", num_cores=NC, num_subcores=NS
)


def _body(idx_ref, w_ref, dsc_ref, out_ref, *sc):
    gidx = sc[0:2]
    didx = sc[2:4]
    rows = sc[4:6]
    gsem = sc[6:8]
    ssem = sc[8:10]

    wid = lax.axis_index("s") * NC + lax.axis_index("c")

    def drain_scatter(p):
        pltpu.make_async_copy(rows[p], out_ref.at[pl.ds(0, HALF)], ssem[p]).wait()

    for u in range(NUNITS):
        f, h = divmod(u, 2)
        p = u & 1
        if u >= 2:
            drain_scatter(p)
        pltpu.sync_copy(idx_ref.at[f, wid, pl.ds(h * HC, HC)], gidx[p])
        pltpu.sync_copy(dsc_ref.at[f, wid, pl.ds(h * HC, HC)], didx[p])

        def gbody(j, carry, f=f, p=p):
            pltpu.async_copy(
                w_ref.at[f].at[gidx[p].at[j]],
                rows[p].at[pl.ds(j * CL, CL)],
                gsem[p],
            )
            return carry

        lax.fori_loop(0, HC, gbody, 0)
        pltpu.make_async_copy(out_ref.at[pl.ds(0, HALF)], rows[p], gsem[p]).wait()

        def sbody(j, carry, p=p):
            pltpu.async_copy(
                rows[p].at[pl.ds(j * CL, CL)],
                out_ref.at[didx[p].at[j]],
                ssem[p],
            )
            return carry

        lax.fori_loop(0, HC, sbody, 0)

    drain_scatter(NUNITS % 2)
    drain_scatter((NUNITS + 1) % 2)


_emb_call = pl.kernel(
    _body,
    out_type=jax.ShapeDtypeStruct((B * NFIELDS * L, EMB), jnp.float32),
    mesh=_mesh,
    compiler_params=pltpu.CompilerParams(use_tc_tiling_on_sc=False),
    scratch_types=[
        pltpu.VMEM((HC, CL), jnp.int32),
        pltpu.VMEM((HC, CL), jnp.int32),
        pltpu.VMEM((HC, CL), jnp.int32),
        pltpu.VMEM((HC, CL), jnp.int32),
        pltpu.VMEM((HALF, EMB), jnp.float32),
        pltpu.VMEM((HALF, EMB), jnp.float32),
        pltpu.SemaphoreType.DMA,
        pltpu.SemaphoreType.DMA,
        pltpu.SemaphoreType.DMA,
        pltpu.SemaphoreType.DMA,
    ],
)


def _dest_rows():
    q = jnp.arange(B * L, dtype=jnp.int32)
    base = (q // L) * (NFIELDS * L) + (q % L)
    dsc = base[None, :] + (jnp.arange(NFIELDS, dtype=jnp.int32) * L)[:, None]
    return dsc.reshape(NFIELDS, NW, 2 * HC, CL)


def kernel(f0, f1, f2, f3, f4, f5, f6, f7, f8, f9, f10, f11, f12, f13, f14,
           f15, f16, f17, f18, f19, f20, f21, f22, f23, f24, f25,
           W_f0, W_f1, W_f2, W_f3, W_f4, W_f5, W_f6, W_f7, W_f8, W_f9, W_f10,
           W_f11, W_f12, W_f13, W_f14, W_f15, W_f16, W_f17, W_f18, W_f19,
           W_f20, W_f21, W_f22, W_f23, W_f24, W_f25):
    idxs = [f0, f1, f2, f3, f4, f5, f6, f7, f8, f9, f10, f11, f12, f13, f14,
            f15, f16, f17, f18, f19, f20, f21, f22, f23, f24, f25]
    tables = [W_f0, W_f1, W_f2, W_f3, W_f4, W_f5, W_f6, W_f7, W_f8, W_f9,
              W_f10, W_f11, W_f12, W_f13, W_f14, W_f15, W_f16, W_f17, W_f18,
              W_f19, W_f20, W_f21, W_f22, W_f23, W_f24, W_f25]
    idx_all = jnp.stack([x.astype(jnp.int32) for x in idxs]).reshape(
        NFIELDS, NW, 2 * HC, CL)
    w_all = jnp.stack(tables)
    out_flat = _emb_call(idx_all, w_all, _dest_rows())
    return out_flat.reshape(B, NFIELDS * L, EMB)


# speedup vs baseline: 2.8439x; 1.1475x over previous
import functools

import jax
import jax.numpy as jnp
from jax import lax
from jax.experimental import pallas as pl
from jax.experimental.pallas import tpu as pltpu
from jax.experimental.pallas import tpu_sc as plsc

NFIELDS = 26
VOCAB = 100000
EMB = 32
B = 4096
L = 20

NC = 2
NS = 16
NW = NC * NS
BPW = B // NW
CL = 128
HC = 10
HALF = HC * CL
NUNITS = 2 * NFIELDS

_mesh = plsc.VectorSubcoreMesh(
    core_axis_name="c", subcore_axis_name="s", num_cores=NC, num_subcores=NS
)


def _body(*refs):
    idx_refs = refs[0:NFIELDS]
    w_refs = refs[NFIELDS:2 * NFIELDS]
    dsc_ref = refs[2 * NFIELDS]
    out_ref = refs[2 * NFIELDS + 1]
    sc = refs[2 * NFIELDS + 2:]
    gidx = sc[0:2]
    didx = sc[2:4]
    rows = sc[4:6]
    gsem = sc[6:8]
    ssem = sc[8:10]

    wid = lax.axis_index("s") * NC + lax.axis_index("c")

    def drain_scatter(p):
        pltpu.make_async_copy(rows[p], out_ref.at[pl.ds(0, HALF)], ssem[p]).wait()

    for u in range(NUNITS):
        f, h = divmod(u, 2)
        p = u & 1
        if u >= 2:
            drain_scatter(p)
        pltpu.sync_copy(idx_refs[f].at[pl.ds(h * HC, HC), wid], gidx[p])
        pltpu.sync_copy(dsc_ref.at[f, pl.ds(h * HC, HC), wid], didx[p])

        def gbody(j, carry, f=f, p=p):
            pltpu.async_copy(
                w_refs[f].at[gidx[p].at[j]],
                rows[p].at[pl.ds(j * CL, CL)],
                gsem[p],
            )
            return carry

        lax.fori_loop(0, HC, gbody, 0)
        pltpu.make_async_copy(out_ref.at[pl.ds(0, HALF)], rows[p], gsem[p]).wait()

        def sbody(j, carry, p=p):
            pltpu.async_copy(
                rows[p].at[pl.ds(j * CL, CL)],
                out_ref.at[didx[p].at[j]],
                ssem[p],
            )
            return carry

        lax.fori_loop(0, HC, sbody, 0)

    drain_scatter(NUNITS % 2)
    drain_scatter((NUNITS + 1) % 2)


_emb_call = pl.kernel(
    _body,
    out_type=jax.ShapeDtypeStruct((B * NFIELDS * L, EMB), jnp.float32),
    mesh=_mesh,
    compiler_params=pltpu.CompilerParams(use_tc_tiling_on_sc=False),
    scratch_types=[
        pltpu.VMEM((HC, CL), jnp.int32),
        pltpu.VMEM((HC, CL), jnp.int32),
        pltpu.VMEM((HC, CL), jnp.int32),
        pltpu.VMEM((HC, CL), jnp.int32),
        pltpu.VMEM((HALF, EMB), jnp.float32),
        pltpu.VMEM((HALF, EMB), jnp.float32),
        pltpu.SemaphoreType.DMA,
        pltpu.SemaphoreType.DMA,
        pltpu.SemaphoreType.DMA,
        pltpu.SemaphoreType.DMA,
    ],
)


def _dest_rows():
    f = jnp.arange(NFIELDS, dtype=jnp.int32)[:, None, None, None]
    l = jnp.arange(L, dtype=jnp.int32)[None, :, None, None]
    w = jnp.arange(NW, dtype=jnp.int32)[None, None, :, None]
    i = jnp.arange(CL, dtype=jnp.int32)[None, None, None, :]
    b = w * BPW + i
    return b * (NFIELDS * L) + f * L + l


def kernel(f0, f1, f2, f3, f4, f5, f6, f7, f8, f9, f10, f11, f12, f13, f14,
           f15, f16, f17, f18, f19, f20, f21, f22, f23, f24, f25,
           W_f0, W_f1, W_f2, W_f3, W_f4, W_f5, W_f6, W_f7, W_f8, W_f9, W_f10,
           W_f11, W_f12, W_f13, W_f14, W_f15, W_f16, W_f17, W_f18, W_f19,
           W_f20, W_f21, W_f22, W_f23, W_f24, W_f25):
    idxs = [f0, f1, f2, f3, f4, f5, f6, f7, f8, f9, f10, f11, f12, f13, f14,
            f15, f16, f17, f18, f19, f20, f21, f22, f23, f24, f25]
    tables = [W_f0, W_f1, W_f2, W_f3, W_f4, W_f5, W_f6, W_f7, W_f8, W_f9,
              W_f10, W_f11, W_f12, W_f13, W_f14, W_f15, W_f16, W_f17, W_f18,
              W_f19, W_f20, W_f21, W_f22, W_f23, W_f24, W_f25]
    idx_t = [x.astype(jnp.int32).T.reshape(L, NW, CL) for x in idxs]
    out_flat = _emb_call(*idx_t, *tables, _dest_rows())
    return out_flat.reshape(B, NFIELDS * L, EMB)
